# baseline (speedup 1.0000x reference)
import jax
import jax.numpy as jnp
from jax import lax
from jax.experimental import pallas as pl
from jax.experimental.pallas import tpu as pltpu
from jax.experimental.pallas import tpu_sc as plsc


def _build_sc_kernel(B, N, SPECIAL, K, D):
    NS = N + SPECIAL
    ROW = NS * D
    NW = 32
    rows_per_w = (B * N) // NW
    brows_per_w = (B * SPECIAL) // NW
    LQ = D // 16

    mesh = plsc.VectorSubcoreMesh(core_axis_name="c", subcore_axis_name="s")

    @jax.jit
    def run(sp_flat, ef_flat, W_edge, W_spd, wv_flat):
        @pl.kernel(
            out_type=jax.ShapeDtypeStruct((B * NS * NS * D,), jnp.float32),
            mesh=mesh,
            compiler_params=pltpu.CompilerParams(use_tc_tiling_on_sc=False),
            scratch_types=[
                pltpu.VMEM((2, N), jnp.int32),
                pltpu.VMEM((2, K * N), jnp.int32),
                pltpu.VMEM((2, N, D), jnp.float32),
                pltpu.VMEM((2, K * N, D), jnp.float32),
                pltpu.VMEM((2, ROW), jnp.float32),
                pltpu.VMEM((ROW,), jnp.float32),
                pltpu.VMEM((D,), jnp.float32),
                pltpu.SemaphoreType.DMA,
                pltpu.SemaphoreType.DMA,
                pltpu.SemaphoreType.DMA,
                pltpu.SemaphoreType.DMA,
                pltpu.SemaphoreType.DMA,
                pltpu.SemaphoreType.DMA,
            ],
        )
        def k(sp_hbm, ef_hbm, we_hbm, ws_hbm, wv_hbm, out_hbm,
              spv2, efv2, bufs2, bufe2, acc2, vrow, vnv,
              sem_i0, sem_i1, sem_g0, sem_g1, sem_o0, sem_o1):
            wid = lax.axis_index("s") * 2 + lax.axis_index("c")
            sem_i = (sem_i0, sem_i1)
            sem_g = (sem_g0, sem_g1)
            sem_o = (sem_o0, sem_o1)

            def row_bi(g):
                r = wid * rows_per_w + g
                b = r // N
                return b, r - b * N

            def issue_idx(g, p):
                r = wid * rows_per_w + g
                pltpu.async_copy(sp_hbm.at[pl.ds(r * N, N)], spv2.at[p], sem_i[p])
                pltpu.async_copy(ef_hbm.at[pl.ds(r * K * N, K * N)], efv2.at[p],
                                 sem_i[p])

            def wait_idx(p):
                pltpu.make_async_copy(sp_hbm.at[pl.ds(0, N)], spv2.at[p],
                                      sem_i[p]).wait()
                pltpu.make_async_copy(ef_hbm.at[pl.ds(0, K * N)], efv2.at[p],
                                      sem_i[p]).wait()

            def issue_gathers(p):
                pltpu.async_copy(ws_hbm.at[spv2.at[p]], bufs2.at[p], sem_g[p])
                for c in range(K):
                    pltpu.async_copy(we_hbm.at[efv2.at[p].at[pl.ds(c * N, N)]],
                                     bufe2.at[p].at[pl.ds(c * N, N)], sem_g[p])

            def wait_gathers(p):
                pltpu.make_async_copy(ws_hbm.at[spv2.at[p]], bufs2.at[p],
                                      sem_g[p]).wait()
                for c in range(K):
                    pltpu.make_async_copy(we_hbm.at[efv2.at[p].at[pl.ds(c * N, N)]],
                                          bufe2.at[p].at[pl.ds(c * N, N)],
                                          sem_g[p]).wait()

            def combine_out(g, p):
                bufe = bufe2.at[p]
                bufs = bufs2.at[p]
                acc = acc2.at[p]

                @plsc.parallel_loop(0, N, unroll=4)
                def _(j):
                    t0 = j * K
                    a = (j + SPECIAL) * D
                    for q in range(LQ):
                        s = pl.ds(q * 16, 16)
                        e = bufe[t0, s] + bufe[t0 + 1, s] + bufe[t0 + 2, s]
                        acc[pl.ds(a + q * 16, 16)] = bufs[j, s] + e * (1.0 / 3.0)

                b, i = row_bi(g)
                pltpu.async_copy(acc2.at[p],
                                 out_hbm.at[pl.ds((b * NS + i + SPECIAL) * ROW, ROW)],
                                 sem_o[p])

            def wait_out(p):
                pltpu.make_async_copy(acc2.at[p], out_hbm.at[pl.ds(0, ROW)],
                                      sem_o[p]).wait()

            pltpu.async_copy(wv_hbm, vnv, sem_i0).wait()

            @plsc.parallel_loop(0, NS, unroll=4)
            def _(r):
                for q in range(LQ):
                    s = pl.ds(q * 16, 16)
                    vrow[pl.ds(r * D + q * 16, 16)] = vnv[s]

            @plsc.parallel_loop(0, SPECIAL)
            def _(r):
                for q in range(LQ):
                    s = pl.ds(q * 16, 16)
                    acc2[0, pl.ds(r * D + q * 16, 16)] = vnv[s]
                    acc2[1, pl.ds(r * D + q * 16, 16)] = vnv[s]

            for p in range(brows_per_w):
                r2 = wid * brows_per_w + p
                b2 = r2 // SPECIAL
                i2 = r2 - b2 * SPECIAL
                pltpu.async_copy(vrow, out_hbm.at[pl.ds((b2 * NS + i2) * ROW, ROW)],
                                 sem_o[p])

            issue_idx(0, 0)
            issue_idx(1, 1)
            wait_idx(0)
            issue_gathers(0)

            @pl.loop(0, rows_per_w, step=2)
            def _(g0):
                g2 = (g0 + 2) % rows_per_w
                g3 = (g0 + 3) % rows_per_w
                wait_idx(1)
                issue_gathers(1)
                wait_gathers(0)
                issue_idx(g2, 0)
                wait_out(0)
                combine_out(g0, 0)
                wait_gathers(1)
                wait_idx(0)
                issue_gathers(0)
                issue_idx(g3, 1)
                wait_out(1)
                combine_out(g0 + 1, 1)

            wait_gathers(0)
            wait_idx(1)
            wait_out(0)
            wait_out(1)

        return k(sp_flat, ef_flat, W_edge, W_spd, wv_flat)

    return run


def kernel(shortest_path, edge_feat, seg_feat, graph_attn_bias, W_edge, W_spd, W_vnode):
    B, N, _ = shortest_path.shape
    K = edge_feat.shape[-1]
    SPECIAL = 2 + seg_feat.shape[1]
    D = W_vnode.shape[-1]
    NS = N + SPECIAL
    run = _build_sc_kernel(B, N, SPECIAL, K, D)
    out_flat = run(shortest_path.reshape(-1), edge_feat.reshape(-1),
                   W_edge, W_spd, W_vnode.reshape(-1))
    return out_flat.reshape(B, NS, NS, D)


# speedup vs baseline: 1.6988x; 1.6988x over previous
import jax
import jax.numpy as jnp
from jax import lax
from jax.experimental import pallas as pl
from jax.experimental.pallas import tpu as pltpu
from jax.experimental.pallas import tpu_sc as plsc


def _build_sc_kernel(B, N, SPECIAL, K, D, NUM_EDGE, NUM_SPATIAL):
    NS = N + SPECIAL
    NW = 32
    rows_per_w = (B * N) // NW
    brows_per_w = (B * SPECIAL) // NW
    LQ = D // 16

    mesh = plsc.VectorSubcoreMesh(core_axis_name="c", subcore_axis_name="s")

    @jax.jit
    def run(sp, ef4, W_edge, W_spd, W_vnode):
        @pl.kernel(
            out_type=jax.ShapeDtypeStruct((B, NS, NS, D), jnp.float32),
            mesh=mesh,
            compiler_params=pltpu.CompilerParams(use_tc_tiling_on_sc=False),
            scratch_types=[
                pltpu.VMEM((2, N), jnp.int32),
                pltpu.VMEM((2, K, N), jnp.int32),
                pltpu.VMEM((2, N, D), jnp.float32),
                pltpu.VMEM((2, K * N, D), jnp.float32),
                pltpu.VMEM((2, NS, D), jnp.float32),
                pltpu.VMEM((NS, D), jnp.float32),
                pltpu.VMEM((1, D), jnp.float32),
                pltpu.VMEM_SHARED((NUM_SPATIAL, D), jnp.float32),
                pltpu.VMEM_SHARED((NUM_EDGE, D), jnp.float32),
                pltpu.SemaphoreType.DMA,
                pltpu.SemaphoreType.DMA,
                pltpu.SemaphoreType.DMA,
                pltpu.SemaphoreType.DMA,
                pltpu.SemaphoreType.DMA,
                pltpu.SemaphoreType.DMA,
            ],
        )
        def k(sp_hbm, ef_hbm, we_hbm, ws_hbm, wv_hbm, out_hbm,
              spv2, efv2, bufs2, bufe2, acc2, vrow, vnv, ws_sh, we_sh,
              sem_i0, sem_i1, sem_g0, sem_g1, sem_o0, sem_o1):
            sid = lax.axis_index("s")
            wid = sid * 2 + lax.axis_index("c")
            sem_i = (sem_i0, sem_i1)
            sem_g = (sem_g0, sem_g1)
            sem_o = (sem_o0, sem_o1)

            def row_bi(g):
                r = wid * rows_per_w + g
                b = r // N
                return b, r - b * N

            def issue_idx(g, p):
                b, i = row_bi(g)
                pltpu.async_copy(sp_hbm.at[b, i], spv2.at[p], sem_i[p])
                pltpu.async_copy(ef_hbm.at[b, i], efv2.at[p], sem_i[p])

            def wait_idx(p):
                pltpu.make_async_copy(sp_hbm.at[0, 0], spv2.at[p], sem_i[p]).wait()
                pltpu.make_async_copy(ef_hbm.at[0, 0], efv2.at[p], sem_i[p]).wait()

            def issue_gathers(p):
                pltpu.async_copy(ws_sh.at[spv2.at[p]], bufs2.at[p], sem_g[p])
                for c in range(K):
                    pltpu.async_copy(we_sh.at[efv2.at[p].at[c]],
                                     bufe2.at[p].at[pl.ds(c * N, N)], sem_g[p])

            def wait_gathers(p):
                pltpu.make_async_copy(ws_sh.at[spv2.at[p]], bufs2.at[p],
                                      sem_g[p]).wait()
                for c in range(K):
                    pltpu.make_async_copy(we_sh.at[efv2.at[p].at[c]],
                                          bufe2.at[p].at[pl.ds(c * N, N)],
                                          sem_g[p]).wait()

            def combine_out(g, p):
                bufe = bufe2.at[p]
                bufs = bufs2.at[p]
                acc = acc2.at[p]

                @plsc.parallel_loop(0, N, unroll=4)
                def _(j):
                    t0 = j * K
                    a = j + SPECIAL
                    for q in range(LQ):
                        s = pl.ds(q * 16, 16)
                        e = bufe[t0, s] + bufe[t0 + 1, s] + bufe[t0 + 2, s]
                        acc[a, s] = bufs[j, s] + e * (1.0 / 3.0)

                b, i = row_bi(g)
                pltpu.async_copy(acc2.at[p], out_hbm.at[b, i + SPECIAL], sem_o[p])

            def wait_out(p):
                pltpu.make_async_copy(acc2.at[p], out_hbm.at[0, 0], sem_o[p]).wait()

            pltpu.sync_copy(ws_hbm.at[pl.ds(sid * (NUM_SPATIAL // 16), NUM_SPATIAL // 16)],
                            ws_sh.at[pl.ds(sid * (NUM_SPATIAL // 16), NUM_SPATIAL // 16)])
            pltpu.sync_copy(we_hbm.at[pl.ds(sid * (NUM_EDGE // 16), NUM_EDGE // 16)],
                            we_sh.at[pl.ds(sid * (NUM_EDGE // 16), NUM_EDGE // 16)])
            plsc.subcore_barrier()

            pltpu.async_copy(wv_hbm, vnv, sem_i0).wait()

            @plsc.parallel_loop(0, NS, unroll=4)
            def _(r):
                for q in range(LQ):
                    s = pl.ds(q * 16, 16)
                    vrow[r, s] = vnv[0, s]

            @plsc.parallel_loop(0, SPECIAL)
            def _(r):
                for q in range(LQ):
                    s = pl.ds(q * 16, 16)
                    acc2[0, r, s] = vnv[0, s]
                    acc2[1, r, s] = vnv[0, s]

            for p in range(brows_per_w):
                r2 = wid * brows_per_w + p
                b2 = r2 // SPECIAL
                i2 = r2 - b2 * SPECIAL
                pltpu.async_copy(vrow, out_hbm.at[b2, i2], sem_o[p])

            issue_idx(0, 0)
            issue_idx(1, 1)
            wait_idx(0)
            issue_gathers(0)

            @pl.loop(0, rows_per_w, step=2)
            def _(g0):
                g2 = (g0 + 2) % rows_per_w
                g3 = (g0 + 3) % rows_per_w
                wait_idx(1)
                issue_gathers(1)
                wait_gathers(0)
                issue_idx(g2, 0)
                wait_out(0)
                combine_out(g0, 0)
                wait_gathers(1)
                wait_idx(0)
                issue_gathers(0)
                issue_idx(g3, 1)
                wait_out(1)
                combine_out(g0 + 1, 1)

            wait_gathers(0)
            wait_idx(1)
            wait_out(0)
            wait_out(1)

        return k(sp, ef4, W_edge, W_spd, W_vnode)

    return run


def kernel(shortest_path, edge_feat, seg_feat, graph_attn_bias, W_edge, W_spd, W_vnode):
    B, N, _ = shortest_path.shape
    K = edge_feat.shape[-1]
    SPECIAL = 2 + seg_feat.shape[1]
    D = W_vnode.shape[-1]
    ef4 = edge_feat.reshape(B, N, K, N)
    run = _build_sc_kernel(B, N, SPECIAL, K, D, W_edge.shape[0], W_spd.shape[0])
    return run(shortest_path, ef4, W_edge, W_spd, W_vnode)
